# baseline (speedup 1.0000x reference)
import functools

import jax
import jax.numpy as jnp
import numpy as np
from jax import lax
from jax.experimental import pallas as pl
from jax.experimental.pallas import tpu as pltpu
from jax.experimental.pallas import tpu_sc as plsc

N = 50000
L = 32
NNB = L - 1
PI = 3.14159265358
H = float(((2.0 * np.pi) ** 3 / N * L / np.pi / (4.0 / 3.0)) ** (1.0 / 3.0))

NP = 51200
CH = 3200
NCH = NP // CH
NGRP = CH // 128
NPG = NP // 128

BN = 4096
NB = (N + BN - 1) // BN
BG = BN // 128

BNT = 2048
NBT = NP // BNT


def _idx_t_body(n_ref, o_ref):
    o_ref[...] = jnp.clip(jnp.transpose(n_ref[...]), 0, N - 1)


def _idx_transpose(neighbor):
    return pl.pallas_call(
        _idx_t_body,
        grid=(NBT,),
        in_specs=[pl.BlockSpec((BNT, L), lambda b: (b, 0))],
        out_specs=pl.BlockSpec((L, BNT), lambda b: (0, b)),
        out_shape=jax.ShapeDtypeStruct((L, NP), jnp.int32),
    )(neighbor)


def _sc_gather(table, idx):
    mesh = plsc.VectorSubcoreMesh(core_axis_name="c", subcore_axis_name="s")

    @functools.partial(
        pl.kernel,
        out_type=jax.ShapeDtypeStruct((NNB, NPG, 8, 128), jnp.float32),
        mesh=mesh,
        scratch_types=[
            pltpu.VMEM((2, CH), jnp.int32),
            pltpu.VMEM((2, CH, 8), jnp.float32),
            pltpu.VMEM((2, NGRP, 8, 128), jnp.float32),
            pltpu.SemaphoreType.DMA,
            pltpu.SemaphoreType.DMA,
        ],
        compiler_params=pltpu.CompilerParams(use_tc_tiling_on_sc=False,
                                             needs_layout_passes=False),
    )
    def k(table_hbm, idx_hbm, out_hbm, idx_v, rows_v, rows_t, gsem, wsem):
        wid = lax.axis_index("s") * 2 + lax.axis_index("c")

        @pl.when(wid < NNB)
        def _():
            lane = lax.iota(jnp.int32, 16)

            def start(c, p):
                pltpu.sync_copy(
                    idx_hbm.at[pl.ds((wid + 1) * NP + c * CH, CH)],
                    idx_v.at[p])
                return pltpu.async_copy(
                    table_hbm.at[idx_v.at[p]], rows_v.at[p], gsem)

            def transpose(p):
                def grp_body(t, carry2):
                    g = t // 8
                    j0 = (t % 8) * 16
                    row_idx = t * 16 + lane
                    for kk in range(7):
                        col_idx = jnp.full((16,), kk, jnp.int32)
                        vals = plsc.load_gather(rows_v.at[p],
                                                [row_idx, col_idx])
                        rows_t[p, g, kk, pl.ds(j0, 16)] = vals
                    return carry2

                lax.fori_loop(0, CH // 16, grp_body, 0)

            cps = [None] * NCH
            wrs = [None] * NCH
            cps[0] = start(0, 0)
            for c in range(NCH):
                p = c % 2
                if c + 1 < NCH:
                    cps[c + 1] = start(c + 1, 1 - p)
                cps[c].wait()
                if c >= 2:
                    wrs[c - 2].wait()
                transpose(p)
                wrs[c] = pltpu.async_copy(
                    rows_t.at[p], out_hbm.at[wid, pl.ds(c * NGRP, NGRP)],
                    wsem)
            wrs[NCH - 2].wait()
            wrs[NCH - 1].wait()

    return k(table, idx)


def _tc_body(g_ref, t_ref, w1_ref, b1_ref, w2_ref, b2_ref, w3_ref, b3_ref,
             w4_ref, b4_ref, s_ref, o_ref):
    i = pl.program_id(1)
    gi = g_ref[0]
    g = jnp.concatenate([gi[tt] for tt in range(BG)], axis=1)
    t = t_ref[...]

    d = t[0:3] - g[0:3]
    temp1 = jnp.abs(d)
    sgn = -jnp.sign(d) * jnp.sign(d + PI) * jnp.sign(d - PI)
    out = sgn * jnp.minimum(temp1, 2.0 * PI - temp1) / H
    outv = t[3:6] - g[3:6]
    po = out * out
    pv = outv * outv
    pc = out * outv
    out2 = po[0:1] + po[1:2] + po[2:3]
    outv2 = pv[0:1] + pv[1:2] + pv[2:3]
    out2v = pc[0:1] + pc[1:2] + pc[2:3]
    drho1 = t[6:7]
    drho2 = g[6:7]
    sq_out2 = jnp.sqrt(out2)
    sq_outv2 = jnp.sqrt(outv2)

    feat = jnp.concatenate(
        [drho1, drho2, sq_out2, sq_outv2, out2v,
         jnp.zeros((3, BN), jnp.float32)], axis=0)
    r = drho1 - drho2
    dis0 = r / jnp.abs(r)
    disA = out / sq_out2
    disB = outv / sq_outv2

    h1 = jnp.tanh(jnp.dot(w1_ref[...], feat,
                          preferred_element_type=jnp.float32) + b1_ref[...])
    h2 = jnp.tanh(jnp.dot(w2_ref[...], h1.astype(jnp.bfloat16),
                          preferred_element_type=jnp.float32) + b2_ref[...])
    h3 = jnp.tanh(jnp.dot(w3_ref[...], h2.astype(jnp.bfloat16),
                          preferred_element_type=jnp.float32) + b3_ref[...])
    h4 = jnp.dot(w4_ref[...], h3,
                 preferred_element_type=jnp.float32) + b4_ref[...]

    drho0 = h4[2:3] * dis0
    drhov = h4[0:1] * disA + h4[1:2] * disB

    a1 = jnp.abs(s_ref[0])
    a2 = jnp.abs(s_ref[1])
    bt1 = jnp.abs(s_ref[2])
    bt2 = jnp.abs(s_ref[3])
    denom = out2 + 0.1 * H * H
    out_rho = r * (H * H) / denom
    out_rho = -(bt1 + bt2 * jnp.abs(out_rho)) * out_rho
    o = -1.0 * H * jnp.tanh(-1.0 * out2v) / denom
    o = -a1 * o + a2 * o * o
    drho0 = drho0 + out_rho
    drhov = drhov + o * disA

    delta = jnp.concatenate(
        [drho0, drhov, jnp.zeros((4, BN), jnp.float32)], axis=0)

    @pl.when(i == 0)
    def _():
        o_ref[...] = delta

    @pl.when(i != 0)
    def _():
        o_ref[...] = o_ref[...] + delta


def kernel(X, V, rho, W1, b1, W2, b2, W3, b3, W4, b4, W1r, b1r, W2r, b2r,
           W3r, b3r, W4r, b4r, alpha1, alpha2, beta1, beta2, neighbor, batch):
    del batch

    table = jnp.concatenate([X, V, rho, jnp.zeros((N, 1), jnp.float32)],
                            axis=1)
    idx = _idx_transpose(neighbor).reshape(-1)

    g = _sc_gather(table, idx)
    table_t = jnp.transpose(table)

    w1c = jnp.zeros((40, 8), jnp.float32)
    w1c = w1c.at[0:20, 0:5].set(W1.T).at[20:40, 0:5].set(W1r.T)
    b1c = jnp.concatenate([b1, b1r]).reshape(40, 1)
    w2c = jnp.zeros((200, 40), jnp.float32)
    w2c = w2c.at[0:100, 0:20].set(W2.T).at[100:200, 20:40].set(W2r.T)
    w2c = w2c.astype(jnp.bfloat16)
    b2c = jnp.concatenate([b2, b2r]).reshape(200, 1)
    w3c = jnp.zeros((40, 200), jnp.float32)
    w3c = w3c.at[0:20, 0:100].set(W3.T).at[20:40, 100:200].set(W3r.T)
    w3c = w3c.astype(jnp.bfloat16)
    b3c = jnp.concatenate([b3, b3r]).reshape(40, 1)
    w4c = jnp.zeros((8, 40), jnp.float32)
    w4c = w4c.at[0:2, 0:20].set(W4.T).at[2:3, 20:40].set(W4r.T)
    b4c = jnp.zeros((8, 1), jnp.float32)
    b4c = b4c.at[0:2, 0].set(b4).at[2, 0].set(b4r[0])
    scal = jnp.stack([alpha1, alpha2, beta1, beta2])

    out_t = pl.pallas_call(
        _tc_body,
        grid=(NB, NNB),
        in_specs=[
            pl.BlockSpec((1, BG, 8, 128), lambda b, i: (i, b, 0, 0)),
            pl.BlockSpec((8, BN), lambda b, i: (0, b)),
            pl.BlockSpec((40, 8), lambda b, i: (0, 0)),
            pl.BlockSpec((40, 1), lambda b, i: (0, 0)),
            pl.BlockSpec((200, 40), lambda b, i: (0, 0)),
            pl.BlockSpec((200, 1), lambda b, i: (0, 0)),
            pl.BlockSpec((40, 200), lambda b, i: (0, 0)),
            pl.BlockSpec((40, 1), lambda b, i: (0, 0)),
            pl.BlockSpec((8, 40), lambda b, i: (0, 0)),
            pl.BlockSpec((8, 1), lambda b, i: (0, 0)),
            pl.BlockSpec(memory_space=pltpu.SMEM),
        ],
        out_specs=pl.BlockSpec((8, BN), lambda b, i: (0, b)),
        out_shape=jax.ShapeDtypeStruct((8, N), jnp.float32),
    )(g, table_t, w1c, b1c, w2c, b2c, w3c, b3c, w4c, b4c, scal)
    return jnp.transpose(out_t[0:4])


# speedup vs baseline: 1.0524x; 1.0524x over previous
import functools

import jax
import jax.numpy as jnp
import numpy as np
from jax import lax
from jax.experimental import pallas as pl
from jax.experimental.pallas import tpu as pltpu
from jax.experimental.pallas import tpu_sc as plsc

N = 50000
L = 32
NNB = L - 1
PI = 3.14159265358
H = float(((2.0 * np.pi) ** 3 / N * L / np.pi / (4.0 / 3.0)) ** (1.0 / 3.0))

NP = 51200
CH = 3200
NCH = NP // CH
NGRP = CH // 128
NPG = NP // 128

BN = 8192
NB = (N + BN - 1) // BN
BG = BN // 128

BNT = 2048
NBT = NP // BNT


def _idx_t_body(n_ref, o_ref):
    o_ref[...] = jnp.clip(jnp.transpose(n_ref[...]), 0, N - 1)


def _idx_transpose(neighbor):
    return pl.pallas_call(
        _idx_t_body,
        grid=(NBT,),
        in_specs=[pl.BlockSpec((BNT, L), lambda b: (b, 0))],
        out_specs=pl.BlockSpec((L, BNT), lambda b: (0, b)),
        out_shape=jax.ShapeDtypeStruct((L, NP), jnp.int32),
    )(neighbor)


def _sc_gather(table, idx):
    mesh = plsc.VectorSubcoreMesh(core_axis_name="c", subcore_axis_name="s")

    @functools.partial(
        pl.kernel,
        out_type=jax.ShapeDtypeStruct((NNB, NPG, 8, 128), jnp.float32),
        mesh=mesh,
        scratch_types=[
            pltpu.VMEM((2, CH), jnp.int32),
            pltpu.VMEM((2, CH, 8), jnp.float32),
            pltpu.VMEM((2, NGRP, 8, 128), jnp.float32),
            pltpu.SemaphoreType.DMA,
            pltpu.SemaphoreType.DMA,
        ],
        compiler_params=pltpu.CompilerParams(use_tc_tiling_on_sc=False,
                                             needs_layout_passes=False),
    )
    def k(table_hbm, idx_hbm, out_hbm, idx_v, rows_v, rows_t, gsem, wsem):
        wid = lax.axis_index("s") * 2 + lax.axis_index("c")

        @pl.when(wid < NNB)
        def _():
            lane = lax.iota(jnp.int32, 16)

            def start(c, p):
                pltpu.sync_copy(
                    idx_hbm.at[pl.ds((wid + 1) * NP + c * CH, CH)],
                    idx_v.at[p])
                return pltpu.async_copy(
                    table_hbm.at[idx_v.at[p]], rows_v.at[p], gsem)

            def transpose(p):
                def grp_body(t, carry2):
                    g = t // 8
                    j0 = (t % 8) * 16
                    row_idx = t * 16 + lane
                    for kk in range(7):
                        col_idx = jnp.full((16,), kk, jnp.int32)
                        vals = plsc.load_gather(rows_v.at[p],
                                                [row_idx, col_idx])
                        rows_t[p, g, kk, pl.ds(j0, 16)] = vals
                    return carry2

                lax.fori_loop(0, CH // 16, grp_body, 0)

            cps = [None] * NCH
            wrs = [None] * NCH
            cps[0] = start(0, 0)
            for c in range(NCH):
                p = c % 2
                if c + 1 < NCH:
                    cps[c + 1] = start(c + 1, 1 - p)
                cps[c].wait()
                if c >= 2:
                    wrs[c - 2].wait()
                transpose(p)
                wrs[c] = pltpu.async_copy(
                    rows_t.at[p], out_hbm.at[wid, pl.ds(c * NGRP, NGRP)],
                    wsem)
            wrs[NCH - 2].wait()
            wrs[NCH - 1].wait()

    return k(table, idx)


def _tc_body(g_ref, t_ref, w1_ref, b1_ref, w2_ref, b2_ref, w3_ref, b3_ref,
             w4_ref, b4_ref, s_ref, o_ref):
    i = pl.program_id(1)
    gi = g_ref[0]
    g = jnp.concatenate([gi[tt] for tt in range(BG)], axis=1)
    t = t_ref[...]

    d = t[0:3] - g[0:3]
    temp1 = jnp.abs(d)
    sgn = -jnp.sign(d) * jnp.sign(d + PI) * jnp.sign(d - PI)
    out = sgn * jnp.minimum(temp1, 2.0 * PI - temp1) / H
    outv = t[3:6] - g[3:6]
    po = out * out
    pv = outv * outv
    pc = out * outv
    out2 = po[0:1] + po[1:2] + po[2:3]
    outv2 = pv[0:1] + pv[1:2] + pv[2:3]
    out2v = pc[0:1] + pc[1:2] + pc[2:3]
    drho1 = t[6:7]
    drho2 = g[6:7]
    sq_out2 = jnp.sqrt(out2)
    sq_outv2 = jnp.sqrt(outv2)

    feat = jnp.concatenate(
        [drho1, drho2, sq_out2, sq_outv2, out2v,
         jnp.zeros((3, BN), jnp.float32)], axis=0)
    r = drho1 - drho2
    dis0 = r / jnp.abs(r)
    disA = out / sq_out2
    disB = outv / sq_outv2

    h1 = jnp.tanh(jnp.dot(w1_ref[...], feat,
                          preferred_element_type=jnp.float32) + b1_ref[...])
    h2 = jnp.tanh(jnp.dot(w2_ref[...], h1,
                          preferred_element_type=jnp.float32) + b2_ref[...])
    h3 = jnp.tanh(jnp.dot(w3_ref[...], h2,
                          preferred_element_type=jnp.float32) + b3_ref[...])
    h4 = jnp.dot(w4_ref[...], h3,
                 preferred_element_type=jnp.float32) + b4_ref[...]

    drho0 = h4[2:3] * dis0
    drhov = h4[0:1] * disA + h4[1:2] * disB

    a1 = jnp.abs(s_ref[0])
    a2 = jnp.abs(s_ref[1])
    bt1 = jnp.abs(s_ref[2])
    bt2 = jnp.abs(s_ref[3])
    denom = out2 + 0.1 * H * H
    out_rho = r * (H * H) / denom
    out_rho = -(bt1 + bt2 * jnp.abs(out_rho)) * out_rho
    o = -1.0 * H * jnp.tanh(-1.0 * out2v) / denom
    o = -a1 * o + a2 * o * o
    drho0 = drho0 + out_rho
    drhov = drhov + o * disA

    delta = jnp.concatenate(
        [drho0, drhov, jnp.zeros((4, BN), jnp.float32)], axis=0)

    @pl.when(i == 0)
    def _():
        o_ref[...] = delta

    @pl.when(i != 0)
    def _():
        o_ref[...] = o_ref[...] + delta


def kernel(X, V, rho, W1, b1, W2, b2, W3, b3, W4, b4, W1r, b1r, W2r, b2r,
           W3r, b3r, W4r, b4r, alpha1, alpha2, beta1, beta2, neighbor, batch):
    del batch

    table = jnp.concatenate([X, V, rho, jnp.zeros((N, 1), jnp.float32)],
                            axis=1)
    idx = _idx_transpose(neighbor).reshape(-1)

    g = _sc_gather(table, idx)
    table_t = jnp.transpose(table)

    w1c = jnp.zeros((40, 8), jnp.float32)
    w1c = w1c.at[0:20, 0:5].set(W1.T).at[20:40, 0:5].set(W1r.T)
    b1c = jnp.concatenate([b1, b1r]).reshape(40, 1)
    w2c = jnp.zeros((200, 40), jnp.float32)
    w2c = w2c.at[0:100, 0:20].set(W2.T).at[100:200, 20:40].set(W2r.T)
    b2c = jnp.concatenate([b2, b2r]).reshape(200, 1)
    w3c = jnp.zeros((40, 200), jnp.float32)
    w3c = w3c.at[0:20, 0:100].set(W3.T).at[20:40, 100:200].set(W3r.T)
    b3c = jnp.concatenate([b3, b3r]).reshape(40, 1)
    w4c = jnp.zeros((8, 40), jnp.float32)
    w4c = w4c.at[0:2, 0:20].set(W4.T).at[2:3, 20:40].set(W4r.T)
    b4c = jnp.zeros((8, 1), jnp.float32)
    b4c = b4c.at[0:2, 0].set(b4).at[2, 0].set(b4r[0])
    scal = jnp.stack([alpha1, alpha2, beta1, beta2])

    out_t = pl.pallas_call(
        _tc_body,
        grid=(NB, NNB),
        in_specs=[
            pl.BlockSpec((1, BG, 8, 128), lambda b, i: (i, b, 0, 0)),
            pl.BlockSpec((8, BN), lambda b, i: (0, b)),
            pl.BlockSpec((40, 8), lambda b, i: (0, 0)),
            pl.BlockSpec((40, 1), lambda b, i: (0, 0)),
            pl.BlockSpec((200, 40), lambda b, i: (0, 0)),
            pl.BlockSpec((200, 1), lambda b, i: (0, 0)),
            pl.BlockSpec((40, 200), lambda b, i: (0, 0)),
            pl.BlockSpec((40, 1), lambda b, i: (0, 0)),
            pl.BlockSpec((8, 40), lambda b, i: (0, 0)),
            pl.BlockSpec((8, 1), lambda b, i: (0, 0)),
            pl.BlockSpec(memory_space=pltpu.SMEM),
        ],
        out_specs=pl.BlockSpec((8, BN), lambda b, i: (0, b)),
        out_shape=jax.ShapeDtypeStruct((8, N), jnp.float32),
    )(g, table_t, w1c, b1c, w2c, b2c, w3c, b3c, w4c, b4c, scal)
    return jnp.transpose(out_t[0:4])


# speedup vs baseline: 1.0758x; 1.0223x over previous
import functools

import jax
import jax.numpy as jnp
import numpy as np
from jax import lax
from jax.experimental import pallas as pl
from jax.experimental.pallas import tpu as pltpu
from jax.experimental.pallas import tpu_sc as plsc

N = 50000
L = 32
NNB = L - 1
PI = 3.14159265358
H = float(((2.0 * np.pi) ** 3 / N * L / np.pi / (4.0 / 3.0)) ** (1.0 / 3.0))

NP = 51200
CH = 3200
NCH = NP // CH
NGRP = CH // 128
NPG = NP // 128

BN = 5120
NB = (N + BN - 1) // BN
BG = BN // 128

BNT = 2048
NBT = NP // BNT


def _idx_t_body(n_ref, o_ref):
    o_ref[...] = jnp.clip(jnp.transpose(n_ref[...]), 0, N - 1)


def _idx_transpose(neighbor):
    return pl.pallas_call(
        _idx_t_body,
        grid=(NBT,),
        in_specs=[pl.BlockSpec((BNT, L), lambda b: (b, 0))],
        out_specs=pl.BlockSpec((L, BNT), lambda b: (0, b)),
        out_shape=jax.ShapeDtypeStruct((L, NP), jnp.int32),
    )(neighbor)


def _sc_gather(table, idx):
    mesh = plsc.VectorSubcoreMesh(core_axis_name="c", subcore_axis_name="s")

    @functools.partial(
        pl.kernel,
        out_type=jax.ShapeDtypeStruct((NNB, NPG, 8, 128), jnp.float32),
        mesh=mesh,
        scratch_types=[
            pltpu.VMEM((2, CH), jnp.int32),
            pltpu.VMEM((2, CH, 8), jnp.float32),
            pltpu.VMEM((2, NGRP, 8, 128), jnp.float32),
            pltpu.SemaphoreType.DMA,
            pltpu.SemaphoreType.DMA,
        ],
        compiler_params=pltpu.CompilerParams(use_tc_tiling_on_sc=False,
                                             needs_layout_passes=False),
    )
    def k(table_hbm, idx_hbm, out_hbm, idx_v, rows_v, rows_t, gsem, wsem):
        wid = lax.axis_index("s") * 2 + lax.axis_index("c")

        @pl.when(wid < NNB)
        def _():
            lane = lax.iota(jnp.int32, 16)

            def start(c, p):
                pltpu.sync_copy(
                    idx_hbm.at[pl.ds((wid + 1) * NP + c * CH, CH)],
                    idx_v.at[p])
                return pltpu.async_copy(
                    table_hbm.at[idx_v.at[p]], rows_v.at[p], gsem)

            def transpose(p):
                def grp_body(t, carry2):
                    g = t // 8
                    j0 = (t % 8) * 16
                    row_idx = t * 16 + lane
                    for kk in range(7):
                        col_idx = jnp.full((16,), kk, jnp.int32)
                        vals = plsc.load_gather(rows_v.at[p],
                                                [row_idx, col_idx])
                        rows_t[p, g, kk, pl.ds(j0, 16)] = vals
                    return carry2

                lax.fori_loop(0, CH // 16, grp_body, 0)

            cps = [None] * NCH
            wrs = [None] * NCH
            cps[0] = start(0, 0)
            for c in range(NCH):
                p = c % 2
                if c + 1 < NCH:
                    cps[c + 1] = start(c + 1, 1 - p)
                cps[c].wait()
                if c >= 2:
                    wrs[c - 2].wait()
                transpose(p)
                wrs[c] = pltpu.async_copy(
                    rows_t.at[p], out_hbm.at[wid, pl.ds(c * NGRP, NGRP)],
                    wsem)
            wrs[NCH - 2].wait()
            wrs[NCH - 1].wait()

    return k(table, idx)


def _tc_body(g_ref, t_ref, w1_ref, b1_ref, w2_ref, b2_ref, w3_ref, b3_ref,
             w4_ref, b4_ref, s_ref, o_ref):
    i = pl.program_id(1)
    gi = g_ref[0]
    g = jnp.concatenate([gi[tt] for tt in range(BG)], axis=1)
    t = t_ref[...]

    d = t[0:3] - g[0:3]
    temp1 = jnp.abs(d)
    sgn = -jnp.sign(d) * jnp.sign(d + PI) * jnp.sign(d - PI)
    out = sgn * jnp.minimum(temp1, 2.0 * PI - temp1) / H
    outv = t[3:6] - g[3:6]
    po = out * out
    pv = outv * outv
    pc = out * outv
    out2 = po[0:1] + po[1:2] + po[2:3]
    outv2 = pv[0:1] + pv[1:2] + pv[2:3]
    out2v = pc[0:1] + pc[1:2] + pc[2:3]
    drho1 = t[6:7]
    drho2 = g[6:7]
    sq_out2 = jnp.sqrt(out2)
    sq_outv2 = jnp.sqrt(outv2)

    feat = jnp.concatenate(
        [drho1, drho2, sq_out2, sq_outv2, out2v,
         jnp.zeros((3, BN), jnp.float32)], axis=0)
    r = drho1 - drho2
    dis0 = r / jnp.abs(r)
    disA = out / sq_out2
    disB = outv / sq_outv2

    h1 = jnp.tanh(jnp.dot(w1_ref[...], feat,
                          preferred_element_type=jnp.float32) + b1_ref[...])
    h2 = jnp.tanh(jnp.dot(w2_ref[...], h1,
                          preferred_element_type=jnp.float32) + b2_ref[...])
    h3 = jnp.tanh(jnp.dot(w3_ref[...], h2,
                          preferred_element_type=jnp.float32) + b3_ref[...])
    h4 = jnp.dot(w4_ref[...], h3,
                 preferred_element_type=jnp.float32) + b4_ref[...]

    drho0 = h4[2:3] * dis0
    drhov = h4[0:1] * disA + h4[1:2] * disB

    a1 = jnp.abs(s_ref[0])
    a2 = jnp.abs(s_ref[1])
    bt1 = jnp.abs(s_ref[2])
    bt2 = jnp.abs(s_ref[3])
    denom = out2 + 0.1 * H * H
    out_rho = r * (H * H) / denom
    out_rho = -(bt1 + bt2 * jnp.abs(out_rho)) * out_rho
    o = -1.0 * H * jnp.tanh(-1.0 * out2v) / denom
    o = -a1 * o + a2 * o * o
    drho0 = drho0 + out_rho
    drhov = drhov + o * disA

    delta = jnp.concatenate(
        [drho0, drhov, jnp.zeros((4, BN), jnp.float32)], axis=0)

    @pl.when(i == 0)
    def _():
        o_ref[...] = delta

    @pl.when(i != 0)
    def _():
        o_ref[...] = o_ref[...] + delta


def kernel(X, V, rho, W1, b1, W2, b2, W3, b3, W4, b4, W1r, b1r, W2r, b2r,
           W3r, b3r, W4r, b4r, alpha1, alpha2, beta1, beta2, neighbor, batch):
    del batch

    table = jnp.concatenate([X, V, rho, jnp.zeros((N, 1), jnp.float32)],
                            axis=1)
    idx = _idx_transpose(neighbor).reshape(-1)

    g = _sc_gather(table, idx)
    table_t = jnp.transpose(table)

    w1c = jnp.zeros((40, 8), jnp.float32)
    w1c = w1c.at[0:20, 0:5].set(W1.T).at[20:40, 0:5].set(W1r.T)
    b1c = jnp.concatenate([b1, b1r]).reshape(40, 1)
    w2c = jnp.zeros((200, 40), jnp.float32)
    w2c = w2c.at[0:100, 0:20].set(W2.T).at[100:200, 20:40].set(W2r.T)
    b2c = jnp.concatenate([b2, b2r]).reshape(200, 1)
    w3c = jnp.zeros((40, 200), jnp.float32)
    w3c = w3c.at[0:20, 0:100].set(W3.T).at[20:40, 100:200].set(W3r.T)
    b3c = jnp.concatenate([b3, b3r]).reshape(40, 1)
    w4c = jnp.zeros((8, 40), jnp.float32)
    w4c = w4c.at[0:2, 0:20].set(W4.T).at[2:3, 20:40].set(W4r.T)
    b4c = jnp.zeros((8, 1), jnp.float32)
    b4c = b4c.at[0:2, 0].set(b4).at[2, 0].set(b4r[0])
    scal = jnp.stack([alpha1, alpha2, beta1, beta2])

    out_t = pl.pallas_call(
        _tc_body,
        grid=(NB, NNB),
        in_specs=[
            pl.BlockSpec((1, BG, 8, 128), lambda b, i: (i, b, 0, 0)),
            pl.BlockSpec((8, BN), lambda b, i: (0, b)),
            pl.BlockSpec((40, 8), lambda b, i: (0, 0)),
            pl.BlockSpec((40, 1), lambda b, i: (0, 0)),
            pl.BlockSpec((200, 40), lambda b, i: (0, 0)),
            pl.BlockSpec((200, 1), lambda b, i: (0, 0)),
            pl.BlockSpec((40, 200), lambda b, i: (0, 0)),
            pl.BlockSpec((40, 1), lambda b, i: (0, 0)),
            pl.BlockSpec((8, 40), lambda b, i: (0, 0)),
            pl.BlockSpec((8, 1), lambda b, i: (0, 0)),
            pl.BlockSpec(memory_space=pltpu.SMEM),
        ],
        out_specs=pl.BlockSpec((8, BN), lambda b, i: (0, b)),
        out_shape=jax.ShapeDtypeStruct((8, N), jnp.float32),
    )(g, table_t, w1c, b1c, w2c, b2c, w3c, b3c, w4c, b4c, scal)
    return jnp.transpose(out_t[0:4])
